# baseline (speedup 1.0000x reference)
import jax
import jax.numpy as jnp
from jax import lax
from jax.experimental import pallas as pl
from jax.experimental.pallas import tpu as pltpu
from jax.experimental.pallas import tpu_sc as plsc

NC, NS = 2, 16
NW = NC * NS
CHUNK = 128
RB = 1024
N_PAD = 49 * RB
F = 16
NBUF = 8
GLA = 4
SUB = 112


def _deg_body(edges_hbm, zeros_hbm, out_hbm, deg_v, idx_v):
    c = lax.axis_index("c")
    s = lax.axis_index("s")
    w = s * NC + c
    nchunks = idx_v.shape[0]
    chunk0 = pl.multiple_of(w * nchunks, 8)
    pltpu.sync_copy(zeros_hbm, deg_v)
    pltpu.sync_copy(edges_hbm.at[1, pl.ds(chunk0, nchunks)], idx_v)
    ones = jnp.ones((16,), jnp.float32)

    def body(r, carry):
        for k in range(8):
            idx16 = idx_v[r, pl.ds(k * 16, 16)]
            plsc.addupdate_scatter(deg_v, [idx16], ones)
        return carry

    lax.fori_loop(0, nchunks, body, 0)
    pltpu.sync_copy(deg_v, out_hbm.at[w])


def _agg_body(edges_hbm, u_hbm, dinv_hbm, crow_hbm, zeros2_hbm, out_hbm,
              acc_sh, src_v, dst_v, rows, gsems, ssems,
              abuf, ubuf, dbuf, crow_v):
    c = lax.axis_index("c")
    s = lax.axis_index("s")
    w = s * NC + c
    rows_per_tile = N_PAD // NS
    nchunks = src_v.shape[0]
    row0 = pl.multiple_of(s * rows_per_tile, 8)
    chunk0 = pl.multiple_of(w * nchunks, 8)
    pltpu.sync_copy(zeros2_hbm.at[pl.ds(row0, rows_per_tile)],
                    acc_sh.at[pl.ds(row0, rows_per_tile)])
    pltpu.sync_copy(edges_hbm.at[0, pl.ds(chunk0, nchunks)], src_v)
    pltpu.sync_copy(edges_hbm.at[1, pl.ds(chunk0, nchunks)], dst_v)
    pltpu.sync_copy(crow_hbm, crow_v)
    plsc.subcore_barrier()

    def gstart(j, b):
        pltpu.async_copy(u_hbm.at[src_v.at[j]], rows[b], gsems[b])

    def gwait(j, b):
        pltpu.make_async_copy(u_hbm.at[src_v.at[j]], rows[b], gsems[b]).wait()

    def sstart(j, b):
        pltpu.async_copy(rows[b], acc_sh.at[dst_v.at[j]], ssems[b], add=True)

    def swait(j, b):
        pltpu.make_async_copy(rows[b], acc_sh.at[dst_v.at[j]], ssems[b]).wait()

    for b in range(GLA):
        gstart(b, b)

    def body(g, carry):
        for b in range(NBUF):
            j = g * NBUF + b
            gwait(j, b)
            sstart(j, b)
            jn = j + GLA
            bn = (b + GLA) % NBUF

            @pl.when(jn < nchunks)
            def _():
                @pl.when(jn >= NBUF)
                def _():
                    swait(jn - NBUF, bn)
                gstart(jn, bn)
        return carry

    lax.fori_loop(0, nchunks // NBUF, body, 0)
    for b in range(NBUF):
        swait(nchunks - NBUF + b, b)

    plsc.subcore_barrier()

    zero16 = jnp.zeros((16,), jnp.float32)
    cr = jnp.where(c == 1, crow_v[pl.ds(0, 16)], zero16)

    def fin_block(o, carry):
        r = row0 + o * SUB
        pltpu.sync_copy(acc_sh.at[pl.ds(r, SUB)], abuf)
        pltpu.sync_copy(dinv_hbm.at[pl.ds(r, SUB)], dbuf)

        @pl.when(c == 1)
        def _():
            pltpu.sync_copy(u_hbm.at[pl.ds(r, SUB)], ubuf)

            def uadd(i, carry2):
                abuf[i] = abuf[i] + ubuf[i]
                return carry2

            lax.fori_loop(0, SUB, uadd, 0)

        def rowloop(i, carry2):
            dv = dbuf[pl.ds(i * 16, 16)]
            for k in range(16):
                row = i * 16 + k
                abuf[row] = dv[k] * abuf[row] + cr
            return carry2

        lax.fori_loop(0, SUB // 16, rowloop, 0)
        pltpu.sync_copy(abuf, out_hbm.at[c, pl.ds(r, SUB)])
        return carry

    lax.fori_loop(0, rows_per_tile // SUB, fin_block, 0)


def _prep_body(x_ref, degp_ref, w2_ref, u_ref, dinv_ref):
    deg = jnp.sum(degp_ref[...], axis=0) + 1.0
    dinv = lax.rsqrt(deg)
    z = jnp.dot(x_ref[...], w2_ref[...], preferred_element_type=jnp.float32)
    u_ref[...] = z * dinv[:, None]
    dinv_ref[...] = dinv


def kernel(nodes, edges, Wg, bg, Wl, bl):
    N = nodes.shape[0]
    E = edges.shape[1]
    x = nodes.reshape(N, 80).astype(jnp.float32)

    Wl3 = Wl.reshape(10, 16, 10)
    W2 = jnp.einsum('cd,kdj->kcj', Wg, Wl3).reshape(80, 10)
    W2p = jnp.zeros((80, F), jnp.float32).at[:, :10].set(W2)
    crow = bg @ Wl3.sum(0) + bl
    crow16 = jnp.zeros((16,), jnp.float32).at[:10].set(crow)

    grp = CHUNK * NW * 8
    E_pad = ((E + grp - 1) // grp) * grp
    nchunks_w = E_pad // (CHUNK * NW)
    edges_m = jnp.pad(edges.astype(jnp.int32), ((0, 0), (0, E_pad - E)),
                      constant_values=N).reshape(2, E_pad // CHUNK, CHUNK)

    zeros1 = jnp.zeros((N_PAD,), jnp.float32)
    zeros2 = jnp.zeros((N_PAD, F), jnp.float32)

    mesh = plsc.VectorSubcoreMesh(core_axis_name="c", subcore_axis_name="s",
                                  num_cores=NC, num_subcores=NS)
    sc_params = pltpu.CompilerParams(needs_layout_passes=False,
                                     use_tc_tiling_on_sc=False)

    degp = pl.kernel(
        _deg_body,
        out_type=jax.ShapeDtypeStruct((NW, N_PAD), jnp.float32),
        mesh=mesh,
        scratch_types=[
            pltpu.VMEM((N_PAD,), jnp.float32),
            pltpu.VMEM((nchunks_w, CHUNK), jnp.int32),
        ],
        compiler_params=sc_params,
    )(edges_m, zeros1)

    nblk = N_PAD // RB
    U, dinv = pl.pallas_call(
        _prep_body,
        grid=(nblk,),
        in_specs=[
            pl.BlockSpec((RB, 80), lambda i: (i, 0)),
            pl.BlockSpec((NW, RB), lambda i: (0, i)),
            pl.BlockSpec((80, F), lambda i: (0, 0)),
        ],
        out_specs=[
            pl.BlockSpec((RB, F), lambda i: (i, 0)),
            pl.BlockSpec((RB,), lambda i: (i,)),
        ],
        out_shape=[
            jax.ShapeDtypeStruct((N_PAD, F), jnp.float32),
            jax.ShapeDtypeStruct((N_PAD,), jnp.float32),
        ],
    )(x, degp, W2p)

    yv = pl.kernel(
        _agg_body,
        out_type=jax.ShapeDtypeStruct((NC, N_PAD, F), jnp.float32),
        mesh=mesh,
        scratch_types=[
            pltpu.VMEM_SHARED((N_PAD, F), jnp.float32),
            pltpu.VMEM((nchunks_w, CHUNK), jnp.int32),
            pltpu.VMEM((nchunks_w, CHUNK), jnp.int32),
            [pltpu.VMEM((CHUNK, F), jnp.float32) for _ in range(NBUF)],
            [pltpu.SemaphoreType.DMA for _ in range(NBUF)],
            [pltpu.SemaphoreType.DMA for _ in range(NBUF)],
            pltpu.VMEM((SUB, F), jnp.float32),
            pltpu.VMEM((SUB, F), jnp.float32),
            pltpu.VMEM((SUB,), jnp.float32),
            pltpu.VMEM((16,), jnp.float32),
        ],
        compiler_params=sc_params,
    )(edges_m, U, dinv, crow16, zeros2)

    return (yv[0] + yv[1])[:N, :10]


# speedup vs baseline: 1.0804x; 1.0804x over previous
import jax
import jax.numpy as jnp
from jax import lax
from jax.experimental import pallas as pl
from jax.experimental.pallas import tpu as pltpu
from jax.experimental.pallas import tpu_sc as plsc

NC, NS = 2, 16
NW = NC * NS
CHUNK = 128
RB = 1024
N_PAD = 49 * RB
F = 16
NBUF = 8
GLA = 4
SUB = 112


def _deg_body(edges_hbm, zeros_hbm, out_hbm, deg_v, idx_v):
    c = lax.axis_index("c")
    s = lax.axis_index("s")
    w = s * NC + c
    nchunks = idx_v.shape[0]
    chunk0 = pl.multiple_of(w * nchunks, 8)
    pltpu.sync_copy(zeros_hbm, deg_v)
    pltpu.sync_copy(edges_hbm.at[1, pl.ds(chunk0, nchunks)], idx_v)
    ones = jnp.ones((16,), jnp.float32)

    def body(r, carry):
        for k in range(8):
            idx16 = idx_v[r, pl.ds(k * 16, 16)]
            plsc.addupdate_scatter(deg_v, [idx16], ones)
        return carry

    lax.fori_loop(0, nchunks, body, 0)
    pltpu.sync_copy(deg_v, out_hbm.at[w])


def _agg_body(edges_hbm, u_hbm, dinv_hbm, crow_hbm, zeros2_hbm, out_hbm,
              acc_sh, src_v, dst_v, rows, gsems, ssems,
              abuf, ubuf, dbuf, crow_v):
    c = lax.axis_index("c")
    s = lax.axis_index("s")
    w = s * NC + c
    rows_per_tile = N_PAD // NS
    nchunks = src_v.shape[0]
    row0 = pl.multiple_of(s * rows_per_tile, 8)
    chunk0 = pl.multiple_of(w * nchunks, 8)
    pltpu.sync_copy(zeros2_hbm.at[pl.ds(row0, rows_per_tile)],
                    acc_sh.at[pl.ds(row0, rows_per_tile)])
    pltpu.sync_copy(edges_hbm.at[0, pl.ds(chunk0, nchunks)], src_v)
    pltpu.sync_copy(edges_hbm.at[1, pl.ds(chunk0, nchunks)], dst_v)
    pltpu.sync_copy(crow_hbm, crow_v)
    plsc.subcore_barrier()

    def gstart(j, b):
        pltpu.async_copy(u_hbm.at[src_v.at[j]], rows[b], gsems[b])

    def gwait(j, b):
        pltpu.make_async_copy(u_hbm.at[src_v.at[j]], rows[b], gsems[b]).wait()

    def sstart(j, b):
        pltpu.async_copy(rows[b], acc_sh.at[dst_v.at[j]], ssems[b], add=True)

    def swait(j, b):
        pltpu.make_async_copy(rows[b], acc_sh.at[dst_v.at[j]], ssems[b]).wait()

    for b in range(GLA):
        gstart(b, b)

    def body(g, carry):
        for b in range(NBUF):
            j = g * NBUF + b
            gwait(j, b)
            sstart(j, b)
            jn = j + GLA
            bn = (b + GLA) % NBUF

            @pl.when(jn < nchunks)
            def _():
                @pl.when(jn >= NBUF)
                def _():
                    swait(jn - NBUF, bn)
                gstart(jn, bn)
        return carry

    lax.fori_loop(0, nchunks // NBUF, body, 0)
    for b in range(NBUF):
        swait(nchunks - NBUF + b, b)

    plsc.subcore_barrier()

    zero16 = jnp.zeros((16,), jnp.float32)
    cr = jnp.where(c == 0, crow_v[pl.ds(0, 16)], zero16)

    def fin_block(o, carry):
        r = row0 + o * SUB
        pltpu.sync_copy(acc_sh.at[pl.ds(r, SUB)], abuf)
        pltpu.sync_copy(dinv_hbm.at[pl.ds(r, SUB)], dbuf)

        @pl.when(c == 0)
        def _():
            pltpu.sync_copy(u_hbm.at[pl.ds(r, SUB)], ubuf)

            def uadd(i, carry2):
                abuf[i] = abuf[i] + ubuf[i]
                return carry2

            lax.fori_loop(0, SUB, uadd, 0)

        def rowloop(i, carry2):
            dv = dbuf[pl.ds(i * 16, 16)]
            for k in range(16):
                row = i * 16 + k
                abuf[row] = dv[k] * abuf[row] + cr
            return carry2

        lax.fori_loop(0, SUB // 16, rowloop, 0)
        pltpu.sync_copy(abuf, out_hbm.at[c, pl.ds(r, SUB)])
        return carry

    lax.fori_loop(0, rows_per_tile // SUB, fin_block, 0)


def _prep_body(x_ref, degp_ref, w2_ref, u_ref, dinv_ref):
    deg = jnp.sum(degp_ref[...], axis=0) + 1.0
    dinv = lax.rsqrt(deg)
    z = jnp.dot(x_ref[...], w2_ref[...], preferred_element_type=jnp.float32)
    u_ref[...] = z * dinv[:, None]
    dinv_ref[...] = dinv


def kernel(nodes, edges, Wg, bg, Wl, bl):
    N = nodes.shape[0]
    E = edges.shape[1]
    x = nodes.reshape(N, 80).astype(jnp.float32)

    Wl3 = Wl.reshape(10, 16, 10)
    W2 = jnp.einsum('cd,kdj->kcj', Wg, Wl3).reshape(80, 10)
    W2p = jnp.zeros((80, F), jnp.float32).at[:, :10].set(W2)
    crow = bg @ Wl3.sum(0) + bl
    crow16 = jnp.zeros((16,), jnp.float32).at[:10].set(crow)

    grp = CHUNK * NW * 8
    E_pad = ((E + grp - 1) // grp) * grp
    nchunks_w = E_pad // (CHUNK * NW)
    edges_m = jnp.pad(edges.astype(jnp.int32), ((0, 0), (0, E_pad - E)),
                      constant_values=N).reshape(2, E_pad // CHUNK, CHUNK)

    zeros1 = jnp.zeros((N_PAD,), jnp.float32)
    zeros2 = jnp.zeros((N_PAD, F), jnp.float32)

    mesh = plsc.VectorSubcoreMesh(core_axis_name="c", subcore_axis_name="s",
                                  num_cores=NC, num_subcores=NS)
    sc_params = pltpu.CompilerParams(needs_layout_passes=False,
                                     use_tc_tiling_on_sc=False)

    degp = pl.kernel(
        _deg_body,
        out_type=jax.ShapeDtypeStruct((NW, N_PAD), jnp.float32),
        mesh=mesh,
        scratch_types=[
            pltpu.VMEM((N_PAD,), jnp.float32),
            pltpu.VMEM((nchunks_w, CHUNK), jnp.int32),
        ],
        compiler_params=sc_params,
    )(edges_m, zeros1)

    nblk = N_PAD // RB
    U, dinv = pl.pallas_call(
        _prep_body,
        grid=(nblk,),
        in_specs=[
            pl.BlockSpec((RB, 80), lambda i: (i, 0)),
            pl.BlockSpec((NW, RB), lambda i: (0, i)),
            pl.BlockSpec((80, F), lambda i: (0, 0)),
        ],
        out_specs=[
            pl.BlockSpec((RB, F), lambda i: (i, 0)),
            pl.BlockSpec((RB,), lambda i: (i,)),
        ],
        out_shape=[
            jax.ShapeDtypeStruct((N_PAD, F), jnp.float32),
            jax.ShapeDtypeStruct((N_PAD,), jnp.float32),
        ],
    )(x, degp, W2p)

    yv = pl.kernel(
        _agg_body,
        out_type=jax.ShapeDtypeStruct((NC, N_PAD, F), jnp.float32),
        mesh=mesh,
        scratch_types=[
            pltpu.VMEM_SHARED((N_PAD, F), jnp.float32),
            pltpu.VMEM((nchunks_w, CHUNK), jnp.int32),
            pltpu.VMEM((nchunks_w, CHUNK), jnp.int32),
            [pltpu.VMEM((CHUNK, F), jnp.float32) for _ in range(NBUF)],
            [pltpu.SemaphoreType.DMA for _ in range(NBUF)],
            [pltpu.SemaphoreType.DMA for _ in range(NBUF)],
            pltpu.VMEM((SUB, F), jnp.float32),
            pltpu.VMEM((SUB, F), jnp.float32),
            pltpu.VMEM((SUB,), jnp.float32),
            pltpu.VMEM((16,), jnp.float32),
        ],
        compiler_params=sc_params,
    )(edges_m, U, dinv, crow16, zeros2)

    return (yv[0] + yv[1])[:N, :10]
